# baseline (speedup 1.0000x reference)
import functools

import jax
import jax.numpy as jnp
from jax import lax
from jax.experimental import pallas as pl
from jax.experimental.pallas import tpu as pltpu
from jax.experimental.pallas import tpu_sc as plsc

_ROWS = 8192
_D = 1024
_NC = 2
_NS = 16
_NW = _NC * _NS
_ROWS_PER_W = _ROWS // _NW


_CHUNK = 32
_NSTEPS = _ROWS_PER_W // _CHUNK
_NBUF = 3
_LAG = 1


@functools.partial(
    pl.kernel,
    mesh=plsc.VectorSubcoreMesh(core_axis_name="c", subcore_axis_name="s"),
    out_type=jax.ShapeDtypeStruct((_ROWS, _D), jnp.float32),
    scratch_types=(
        [pltpu.VMEM((_CHUNK, _D), jnp.float32)] * _NBUF
        + [pltpu.SemaphoreType.DMA] * (2 * _NBUF)
    ),
)
def _sc_copy(table_hbm, out_hbm, *scratch):
    bufs = scratch[:_NBUF]
    sin = scratch[_NBUF : 2 * _NBUF]
    sout = scratch[2 * _NBUF : 3 * _NBUF]
    wid = lax.axis_index("s") * _NC + lax.axis_index("c")
    base = wid * _ROWS_PER_W

    def in_copy(i):
        return pltpu.make_async_copy(
            table_hbm.at[pl.ds(base + i * _CHUNK, _CHUNK)],
            bufs[i % _NBUF],
            sin[i % _NBUF],
        )

    def out_copy(i):
        return pltpu.make_async_copy(
            bufs[i % _NBUF],
            out_hbm.at[pl.ds(base + i * _CHUNK, _CHUNK)],
            sout[i % _NBUF],
        )

    for j in range(_NBUF):
        in_copy(j).start()
    for i in range(_NSTEPS):
        in_copy(i).wait()
        out_copy(i).start()
        j = i - _LAG
        if j >= 0 and j + _NBUF < _NSTEPS:
            out_copy(j).wait()
            in_copy(j + _NBUF).start()
    for j in range(max(0, _NSTEPS - _NBUF), _NSTEPS):
        out_copy(j).wait()


def kernel(x, pos_emb_table):
    out = _sc_copy(pos_emb_table)
    return out[None]


# speedup vs baseline: 1.0171x; 1.0171x over previous
import functools

import jax
import jax.numpy as jnp
from jax import lax
from jax.experimental import pallas as pl
from jax.experimental.pallas import tpu as pltpu
from jax.experimental.pallas import tpu_sc as plsc

_ROWS = 8192
_D = 1024
_NC = 2
_NS = 16
_NW = _NC * _NS
_ROWS_PER_W = _ROWS // _NW


_CHUNK = 8
_NSTEPS = _ROWS_PER_W // _CHUNK
_NBUF = 12
_LAG = 4


@functools.partial(
    pl.kernel,
    mesh=plsc.VectorSubcoreMesh(core_axis_name="c", subcore_axis_name="s"),
    out_type=jax.ShapeDtypeStruct((_ROWS, _D), jnp.float32),
    scratch_types=(
        [pltpu.VMEM((_CHUNK, _D), jnp.float32)] * _NBUF
        + [pltpu.SemaphoreType.DMA] * (2 * _NBUF)
    ),
)
def _sc_copy(table_hbm, out_hbm, *scratch):
    bufs = scratch[:_NBUF]
    sin = scratch[_NBUF : 2 * _NBUF]
    sout = scratch[2 * _NBUF : 3 * _NBUF]
    wid = lax.axis_index("s") * _NC + lax.axis_index("c")
    base = wid * _ROWS_PER_W

    def in_copy(i):
        return pltpu.make_async_copy(
            table_hbm.at[pl.ds(base + i * _CHUNK, _CHUNK)],
            bufs[i % _NBUF],
            sin[i % _NBUF],
        )

    def out_copy(i):
        return pltpu.make_async_copy(
            bufs[i % _NBUF],
            out_hbm.at[pl.ds(base + i * _CHUNK, _CHUNK)],
            sout[i % _NBUF],
        )

    for j in range(_NBUF):
        in_copy(j).start()
    for i in range(_NSTEPS):
        in_copy(i).wait()
        out_copy(i).start()
        j = i - _LAG
        if j >= 0 and j + _NBUF < _NSTEPS:
            out_copy(j).wait()
            in_copy(j + _NBUF).start()
    for j in range(max(0, _NSTEPS - _NBUF), _NSTEPS):
        out_copy(j).wait()


def kernel(x, pos_emb_table):
    out = _sc_copy(pos_emb_table)
    return out[None]


# speedup vs baseline: 1.0314x; 1.0141x over previous
import functools

import jax
import jax.numpy as jnp
from jax import lax
from jax.experimental import pallas as pl
from jax.experimental.pallas import tpu as pltpu
from jax.experimental.pallas import tpu_sc as plsc

_ROWS = 8192
_D = 1024
_NC = 2
_NS = 16
_NW = _NC * _NS
_ROWS_PER_W = _ROWS // _NW


_CHUNK = 16
_NSTEPS = _ROWS_PER_W // _CHUNK
_NBUF = 6
_LAG = 2


@functools.partial(
    pl.kernel,
    mesh=plsc.VectorSubcoreMesh(core_axis_name="c", subcore_axis_name="s"),
    out_type=jax.ShapeDtypeStruct((_ROWS, _D), jnp.float32),
    scratch_types=(
        [pltpu.VMEM((_CHUNK, _D), jnp.float32)] * _NBUF
        + [pltpu.SemaphoreType.DMA] * (2 * _NBUF)
    ),
)
def _sc_copy(table_hbm, out_hbm, *scratch):
    bufs = scratch[:_NBUF]
    sin = scratch[_NBUF : 2 * _NBUF]
    sout = scratch[2 * _NBUF : 3 * _NBUF]
    wid = lax.axis_index("s") * _NC + lax.axis_index("c")
    base = wid * _ROWS_PER_W

    def in_copy(i):
        return pltpu.make_async_copy(
            table_hbm.at[pl.ds(base + i * _CHUNK, _CHUNK)],
            bufs[i % _NBUF],
            sin[i % _NBUF],
        )

    def out_copy(i):
        return pltpu.make_async_copy(
            bufs[i % _NBUF],
            out_hbm.at[pl.ds(base + i * _CHUNK, _CHUNK)],
            sout[i % _NBUF],
        )

    for j in range(_NBUF):
        in_copy(j).start()
    for i in range(_NSTEPS):
        in_copy(i).wait()
        out_copy(i).start()
        j = i - _LAG
        if j >= 0 and j + _NBUF < _NSTEPS:
            out_copy(j).wait()
            in_copy(j + _NBUF).start()
    for j in range(max(0, _NSTEPS - _NBUF), _NSTEPS):
        out_copy(j).wait()


def kernel(x, pos_emb_table):
    out = _sc_copy(pos_emb_table)
    return out[None]
